# baseline (speedup 1.0000x reference)
import jax
import jax.numpy as jnp
from jax.experimental import pallas as pl
from jax.experimental.pallas import tpu as pltpu

_S = 12
_P = 12
_D3 = 48
_BM = 3456
_HALO = 8


def _block_kernel(M, x_ref, xp_ref, xn_ref, x8_ref, w_ref, bqkv_ref,
                  wd_ref, bd_ref, wu_ref, bu_ref, o_ref):
    i = pl.program_id(1)
    BME = _BM + 2 * _HALO
    bf = jnp.bfloat16

    def dott(a, b):
        return jax.lax.dot_general(a, b, (((1,), (1,)), ((), ())),
                                   preferred_element_type=jnp.float32)

    xc = x_ref[0].astype(bf)
    fh = xp_ref[0, _BM - _HALO:, :].astype(bf)
    bh = xn_ref[0, :_HALO, :].astype(bf)
    x8 = x8_ref[0].astype(bf)

    w = w_ref[...]
    wq = w[0:_D3, :]
    wk = w[_D3:2 * _D3, :]
    wv = w[2 * _D3:, :]
    bq = bqkv_ref[0:1, 0:_D3]
    bk = bqkv_ref[0:1, _D3:2 * _D3]
    bv = bqkv_ref[0:1, 2 * _D3:]

    q_ext = jnp.concatenate([dott(fh, wq), dott(xc, wq), dott(bh, wq)],
                            axis=0) + bq
    v_ext = jnp.concatenate([dott(fh, wv), dott(xc, wv), dott(bh, wv)],
                            axis=0) + bv

    rows = jax.lax.broadcasted_iota(jnp.int32, (BME, _D3), 0)
    gi = i * _BM - _HALO + rows
    valid = (gi >= 0) & (gi < M)
    v_m = jnp.where(valid, v_ext, 0.0)

    k8 = dott(x8, wk) + bk
    v8 = dott(x8, wv) + bv
    kd = k8[0:1, :] - k8[1:2, :]
    kd2 = jnp.concatenate([kd, -kd], axis=0).astype(bf)
    qb = q_ext.astype(bf)
    s2 = dott(qb, kd2)
    p2 = jax.nn.sigmoid(s2).astype(bf)
    att = jax.lax.dot_general(p2, v8[0:2, :].astype(bf),
                              (((1,), (0,)), ((), ())),
                              preferred_element_type=jnp.float32)

    wd = wd_ref[...]
    vb = v_m.astype(bf)
    yd0 = dott(vb, wd[:, :, 0])
    yd1 = dott(vb, wd[:, :, 1])
    yd2 = dott(vb, wd[:, :, 2])
    mid_c = att + bd_ref[...] + yd1
    midv = mid_c[1:BME - 1] + yd0[0:BME - 2] + yd2[2:BME]
    midv = jnp.where(valid[1:BME - 1], midv, 0.0).astype(bf)

    wu = wu_ref[...]
    yu0 = dott(midv, wu[:, :, 0])
    yu1 = dott(midv, wu[:, :, 1])
    yu2 = dott(midv, wu[:, :, 2])
    out = (yu0[_HALO - 2:_HALO - 2 + _BM] + yu1[_HALO - 1:_HALO - 1 + _BM]
           + yu2[_HALO:_HALO + _BM] + bu_ref[...])
    o_ref[0] = out.T


def kernel(input, W_qkv, b_qkv, W_down, b_down, W_up, b_up):
    B, C, H, W = input.shape
    xu = input.reshape(B, C, _S, _P, _S, _P)
    xu = jnp.transpose(xu, (0, 1, 3, 5, 2, 4)).reshape(B, C * _P * _P, _S * _S)
    x = xu.reshape(B, _S * _S, -1, _P * _P)
    x = jnp.transpose(x, (0, 2, 1, 3)).reshape(B, -1, _S * _S)
    M = x.shape[1]
    nb = M // _BM

    wqkv = W_qkv.astype(jnp.bfloat16)
    wd = W_down.astype(jnp.bfloat16)
    wu = W_up.astype(jnp.bfloat16)
    bqkv = b_qkv.reshape(1, 3 * _D3)
    bd = b_down.reshape(1, _D3)
    bu = b_up.reshape(1, _D3)

    def full(shp, nd):
        return pl.BlockSpec(shp, (lambda b, i: (0,) * nd))

    out = pl.pallas_call(
        lambda *refs: _block_kernel(M, *refs),
        grid=(B, nb),
        in_specs=[
            pl.BlockSpec((1, _BM, _S * _S), lambda b, i: (b, i, 0)),
            pl.BlockSpec((1, _BM, _S * _S),
                         lambda b, i: (b, jnp.maximum(i - 1, 0), 0)),
            pl.BlockSpec((1, _BM, _S * _S),
                         lambda b, i: (b, jnp.minimum(i + 1, nb - 1), 0)),
            pl.BlockSpec((1, _HALO, _S * _S), lambda b, i: (b, 0, 0)),
            full((_S * _S, _S * _S), 2),
            full((1, 3 * _D3), 2),
            full((_D3, _D3, 3), 3),
            full((1, _D3), 2),
            full((_D3, _D3, 3), 3),
            full((1, _D3), 2),
        ],
        out_specs=pl.BlockSpec((1, _D3, _BM), lambda b, i: (b, 0, i)),
        out_shape=jax.ShapeDtypeStruct((B, _D3, M), jnp.float32),
        compiler_params=pltpu.CompilerParams(
            dimension_semantics=("arbitrary", "arbitrary")),
    )(x, x, x, x, wqkv, bqkv, wd, bd, wu, bu)

    o5 = out.reshape(B, _D3, C // 3, (M // (C // 3)) // 3, 3)
    return jnp.transpose(o5, (0, 3, 2, 4, 1)).reshape(B, C // 3, H, W)
